# baseline (speedup 1.0000x reference)
import functools

import jax
import jax.numpy as jnp
from jax import lax
from jax.experimental import pallas as pl
from jax.experimental.pallas import tpu as pltpu
from jax.experimental.pallas import tpu_sc as plsc

N_NODES = 10000
NP = 10240
D = 128
E = 320000
NPAIR = 8

NS = 16
EPT = 20480
EP = NS * EPT
CHUNK = 1024
NCHUNK = EPT // CHUNK
BLK = 128
BPC = CHUNK // BLK
Q = 2560
QP = 2688
QPT = QP // NS
TPT = NP // NS


def _sc_aggregate(a, b, t, t_rows, zrows):
  mesh = plsc.VectorSubcoreMesh(core_axis_name="c", subcore_axis_name="s")

  @functools.partial(
      pl.kernel,
      out_type=(
          jax.ShapeDtypeStruct((NPAIR, 4 * QP, D), jnp.float32),
          jax.ShapeDtypeStruct((NPAIR, 4 * QP), jnp.float32),
      ),
      mesh=mesh,
      scratch_types=[
          pltpu.VMEM((CHUNK,), jnp.int32),
          pltpu.VMEM((CHUNK,), jnp.int32),
          pltpu.VMEM((CHUNK,), jnp.int32),
          pltpu.VMEM((BPC, BLK), jnp.int32),
          pltpu.VMEM((BLK, D), jnp.float32),
          pltpu.VMEM((128,), jnp.float32),
          pltpu.VMEM((128,), jnp.float32),
          pltpu.SemaphoreType.DMA,
          pltpu.VMEM_SHARED((NP, D), jnp.float32),
          pltpu.VMEM_SHARED((QP, D), jnp.float32),
          pltpu.VMEM_SHARED((QP,), jnp.float32),
      ],
  )
  def k(a_hbm, b_hbm, t_hbm, rows_hbm, z_hbm, g_hbm, cnt_hbm,
        av, bv, tv, dstb, rowbuf, ones_v, zflat,
        csem, t_sh, g_sh, cnt_sh):
    cid = lax.axis_index("c")
    sid = lax.axis_index("s")

    def _zf(i, _):
      zflat[pl.ds(i * 16, 16)] = jnp.zeros((16,), jnp.float32)
      return 0
    lax.fori_loop(0, 8, _zf, 0)

    def _on(i, _):
      ones_v[pl.ds(i * 16, 16)] = jnp.ones((16,), jnp.float32)
      return 0
    lax.fori_loop(0, 8, _on, 0)

    for p in range(4):
      s_glob = 4 * cid + p
      rel = 2 * cid + (p // 2)
      inv = p % 2
      dv0 = bv if inv else av
      sv0 = av if inv else bv

      pltpu.sync_copy(
          rows_hbm.at[pl.ds(s_glob * NP + sid * TPT, TPT)],
          t_sh.at[pl.ds(sid * TPT, TPT)])
      plsc.subcore_barrier()

      for q in range(4):
        qbase = q * Q

        pltpu.sync_copy(z_hbm, g_sh.at[pl.ds(sid * QPT, QPT)])
        pltpu.sync_copy(zflat, cnt_sh.at[pl.ds(sid * 128, 128)])

        @pl.when(sid < QP // 128 - NS)
        def _():
          pltpu.sync_copy(zflat, cnt_sh.at[pl.ds((NS + sid) * 128, 128)])
        plsc.subcore_barrier()

        def chunk_body(c, _):
          start = pl.multiple_of(sid * EPT + c * CHUNK, 128)
          pltpu.sync_copy(a_hbm.at[pl.ds(start, CHUNK)], av)
          pltpu.sync_copy(b_hbm.at[pl.ds(start, CHUNK)], bv)
          pltpu.sync_copy(t_hbm.at[pl.ds(start, CHUNK)], tv)

          def scan16(i, _):
            blk = i // (BLK // 16)
            lane0 = (i % (BLK // 16)) * 16
            t16 = tv[pl.ds(i * 16, 16)]
            d16 = dv0[pl.ds(i * 16, 16)]
            loc = d16 - qbase
            ok = jnp.logical_and(t16 == rel,
                                 jnp.logical_and(loc >= 0, loc < Q))
            dstb[blk, pl.ds(lane0, 16)] = jnp.where(ok, loc, Q)
            return 0

          lax.fori_loop(0, CHUNK // 16, scan16, 0)

          for j in range(BPC):
            pltpu.async_copy(ones_v, cnt_sh.at[dstb.at[j]], csem, add=True)

          def blk_body(j, _):
            pltpu.sync_copy(
                t_sh.at[sv0.at[pl.ds(j * BLK, BLK)]], rowbuf)
            pltpu.sync_copy(rowbuf, g_sh.at[dstb.at[j]], add=True)
            return 0

          lax.fori_loop(0, BPC, blk_body, 0)
          for j in range(BPC):
            pltpu.make_async_copy(ones_v, cnt_sh.at[dstb.at[j]], csem).wait()
          return 0

        lax.fori_loop(0, NCHUNK, chunk_body, 0)
        plsc.subcore_barrier()

        o = sid * QPT
        pltpu.sync_copy(g_sh.at[pl.ds(o, QPT)],
                        g_hbm.at[s_glob].at[pl.ds(q * QP + o, QPT)])
        pltpu.sync_copy(cnt_sh.at[pl.ds(sid * 128, 128)],
                        cnt_hbm.at[s_glob].at[pl.ds(q * QP + sid * 128, 128)])

        @pl.when(sid < QP // 128 - NS)
        def _():
          o2 = (NS + sid) * 128
          pltpu.sync_copy(cnt_sh.at[pl.ds(o2, 128)],
                          cnt_hbm.at[s_glob].at[pl.ds(q * QP + o2, 128)])
        plsc.subcore_barrier()

  return k(a, b, t, t_rows, zrows)


def _transform_body(emb_ref, w_ref, out_ref):
  out_ref[0] = jnp.dot(emb_ref[...], w_ref[0],
                       preferred_element_type=jnp.float32)


def _transform(emb_pad, ws):
  return pl.pallas_call(
      _transform_body,
      grid=(NPAIR, NP // 1024),
      in_specs=[
          pl.BlockSpec((1024, D), lambda s, j: (j, 0)),
          pl.BlockSpec((1, D, D), lambda s, j: (s, 0, 0)),
      ],
      out_specs=pl.BlockSpec((1, 1024, D), lambda s, j: (s, j, 0)),
      out_shape=jax.ShapeDtypeStruct((NPAIR, NP, D), jnp.float32),
  )(emb_pad, ws)


def _combine_body(g_ref, cnt_ref, b_ref, out_ref):
  c = cnt_ref[...]
  bmat = b_ref[...]
  nrows = c.shape[1]
  run = jnp.ones((nrows,), jnp.float32)
  acc = jnp.zeros((nrows, D), jnp.float32)
  for s in range(NPAIR - 1, -1, -1):
    cs = c[s]
    run = run * (2.0 / jnp.maximum(cs, 1.0))
    acc = acc + run[:, None] * g_ref[0, s] + (run * cs)[:, None] * bmat[s][None, :]
  out_ref[...] = acc


def _combine(g, cnt, bs):
  return pl.pallas_call(
      _combine_body,
      grid=(NP // 128,),
      in_specs=[
          pl.BlockSpec((1, NPAIR, 128, D),
                       lambda j: (0, 0, (j // 20) * 21 + j % 20, 0)),
          pl.BlockSpec((NPAIR, 128), lambda j: (0, (j // 20) * 21 + j % 20)),
          pl.BlockSpec((NPAIR, D), lambda j: (0, 0)),
      ],
      out_specs=pl.BlockSpec((128, D), lambda j: (j, 0)),
      out_shape=jax.ShapeDtypeStruct((NP, D), jnp.float32),
  )(g.reshape(1, NPAIR, 4 * QP, D), cnt, bs)


def kernel(edge_index, edge_type, embeddings, W0, b0):
  perm = jnp.array([0, 4, 1, 5, 2, 6, 3, 7], dtype=jnp.int32)
  ws = W0[perm]
  bs = b0[perm]

  emb_pad = jnp.zeros((NP, D), jnp.float32).at[:N_NODES].set(embeddings)
  t_tab = _transform(emb_pad, ws)
  t_rows = t_tab.reshape(NPAIR * NP, D)

  pad = EP - E
  a = jnp.concatenate([edge_index[0], jnp.zeros((pad,), jnp.int32)])
  b = jnp.concatenate([edge_index[1], jnp.zeros((pad,), jnp.int32)])
  t = jnp.concatenate([edge_type, jnp.full((pad,), -1, jnp.int32)])
  zrows = jnp.zeros((QPT, D), jnp.float32)

  g, cnt = _sc_aggregate(a, b, t, t_rows, zrows)
  out = _combine(g, cnt, bs)
  return out[:N_NODES]


# speedup vs baseline: 1.6946x; 1.6946x over previous
import functools

import jax
import jax.numpy as jnp
from jax import lax
from jax.experimental import pallas as pl
from jax.experimental.pallas import tpu as pltpu
from jax.experimental.pallas import tpu_sc as plsc

N_NODES = 10000
NP = 10240
D = 128
E = 320000
NPAIR = 8

NS = 16
EPT = 20480
EP = NS * EPT
CHUNK = 1024
NCHUNK = EPT // CHUNK
BLK = 64
BPC = CHUNK // BLK
Q = 2560
QP = 2688
QPT = QP // NS
TPT = NP // NS


def _sc_aggregate(a, b, t, t_rows, zrows):
  mesh = plsc.VectorSubcoreMesh(core_axis_name="c", subcore_axis_name="s")

  @functools.partial(
      pl.kernel,
      out_type=(
          jax.ShapeDtypeStruct((NPAIR, 4 * QP, D), jnp.float32),
          jax.ShapeDtypeStruct((NPAIR, 4 * QP), jnp.float32),
      ),
      mesh=mesh,
      scratch_types=[
          pltpu.VMEM((CHUNK,), jnp.int32),
          pltpu.VMEM((CHUNK,), jnp.int32),
          pltpu.VMEM((CHUNK,), jnp.int32),
          pltpu.VMEM((BPC, BLK), jnp.int32),
          pltpu.VMEM((BPC, BLK), jnp.int32),
          pltpu.VMEM((2, BLK, D), jnp.float32),
          pltpu.VMEM((CHUNK,), jnp.float32),
          pltpu.VMEM((128,), jnp.float32),
          pltpu.SemaphoreType.DMA,
          pltpu.SemaphoreType.DMA,
          pltpu.SemaphoreType.DMA,
          pltpu.VMEM_SHARED((NP, D), jnp.float32),
          pltpu.VMEM_SHARED((QP, D), jnp.float32),
          pltpu.VMEM_SHARED((NP + 128,), jnp.float32),
      ],
  )
  def k(a_hbm, b_hbm, t_hbm, rows_hbm, z_hbm, g_hbm, cnt_hbm,
        av, bv, tv, dstb, dstc, rowbuf, ones_v, zflat,
        ssem0, ssem1, csem, t_sh, g_sh, cnt_sh):
    cid = lax.axis_index("c")
    sid = lax.axis_index("s")

    def _zf(i, _):
      zflat[pl.ds(i * 16, 16)] = jnp.zeros((16,), jnp.float32)
      return 0
    lax.fori_loop(0, 8, _zf, 0)

    def _on(i, _):
      ones_v[pl.ds(i * 16, 16)] = jnp.ones((16,), jnp.float32)
      return 0
    lax.fori_loop(0, CHUNK // 16, _on, 0)

    for p in range(4):
      s_glob = 4 * cid + p
      rel = 2 * cid + (p // 2)
      inv = p % 2
      dv0 = bv if inv else av
      sv0 = av if inv else bv

      pltpu.sync_copy(
          rows_hbm.at[pl.ds(s_glob * NP + sid * TPT, TPT)],
          t_sh.at[pl.ds(sid * TPT, TPT)])
      for rr in range(6):
        @pl.when(rr * NS + sid < (NP + 128) // 128)
        def _():
          pltpu.sync_copy(
              zflat, cnt_sh.at[pl.ds((rr * NS + sid) * 128, 128)])
      plsc.subcore_barrier()

      for q in range(4):
        qbase = q * Q

        pltpu.sync_copy(z_hbm, g_sh.at[pl.ds(sid * QPT, QPT)])
        plsc.subcore_barrier()

        def chunk_body(c, _):
          start = pl.multiple_of(sid * EPT + c * CHUNK, 128)
          pltpu.sync_copy(a_hbm.at[pl.ds(start, CHUNK)], av)
          pltpu.sync_copy(b_hbm.at[pl.ds(start, CHUNK)], bv)
          pltpu.sync_copy(t_hbm.at[pl.ds(start, CHUNK)], tv)

          def scan16(i, _):
            blk = i // (BLK // 16)
            lane0 = (i % (BLK // 16)) * 16
            t16 = tv[pl.ds(i * 16, 16)]
            d16 = dv0[pl.ds(i * 16, 16)]
            loc = d16 - qbase
            ok = jnp.logical_and(t16 == rel,
                                 jnp.logical_and(loc >= 0, loc < Q))
            dstb[blk, pl.ds(lane0, 16)] = jnp.where(ok, loc, Q)
            if q == 0:
              dstc[blk, pl.ds(lane0, 16)] = jnp.where(t16 == rel, d16, NP)
            return 0

          lax.fori_loop(0, CHUNK // 16, scan16, 0)

          if q == 0:
            for j in range(BPC):
              pltpu.async_copy(ones_v.at[pl.ds(0, BLK)],
                               cnt_sh.at[dstc.at[j]], csem, add=True)

          pltpu.sync_copy(t_sh.at[sv0.at[pl.ds(0, BLK)]], rowbuf.at[0])
          pltpu.sync_copy(t_sh.at[sv0.at[pl.ds(BLK, BLK)]], rowbuf.at[1])
          pltpu.async_copy(rowbuf.at[0], g_sh.at[dstb.at[0]], ssem0,
                           add=True)
          pltpu.async_copy(rowbuf.at[1], g_sh.at[dstb.at[1]], ssem1,
                           add=True)

          def blk_pair(h, _):
            j0 = 2 + h * 2
            pltpu.make_async_copy(rowbuf.at[0], g_sh.at[dstb.at[0]],
                                  ssem0).wait()
            pltpu.sync_copy(
                t_sh.at[sv0.at[pl.ds(j0 * BLK, BLK)]], rowbuf.at[0])
            pltpu.async_copy(rowbuf.at[0], g_sh.at[dstb.at[j0]], ssem0,
                             add=True)
            pltpu.make_async_copy(rowbuf.at[1], g_sh.at[dstb.at[1]],
                                  ssem1).wait()
            pltpu.sync_copy(
                t_sh.at[sv0.at[pl.ds((j0 + 1) * BLK, BLK)]], rowbuf.at[1])
            pltpu.async_copy(rowbuf.at[1], g_sh.at[dstb.at[j0 + 1]], ssem1,
                             add=True)
            return 0

          lax.fori_loop(0, (BPC - 2) // 2, blk_pair, 0)
          pltpu.make_async_copy(rowbuf.at[0], g_sh.at[dstb.at[0]],
                                ssem0).wait()
          pltpu.make_async_copy(rowbuf.at[1], g_sh.at[dstb.at[1]],
                                ssem1).wait()
          if q == 0:
            for j in range(BPC):
              pltpu.make_async_copy(ones_v.at[pl.ds(0, BLK)],
                                    cnt_sh.at[dstc.at[j]], csem).wait()
          return 0

        lax.fori_loop(0, NCHUNK, chunk_body, 0)
        plsc.subcore_barrier()

        o = sid * QPT
        pltpu.sync_copy(g_sh.at[pl.ds(o, QPT)],
                        g_hbm.at[s_glob].at[pl.ds(q * QP + o, QPT)])
        if q == 0:
          for rr in range(5):
            @pl.when(rr * NS + sid < NP // 128)
            def _():
              bb = rr * NS + sid
              pltpu.sync_copy(
                  cnt_sh.at[pl.ds(bb * 128, 128)],
                  cnt_hbm.at[s_glob].at[pl.ds((bb + bb // 20) * 128, 128)])
        plsc.subcore_barrier()

  return k(a, b, t, t_rows, zrows)


def _transform_body(emb_ref, w_ref, out_ref):
  out_ref[0] = jnp.dot(emb_ref[...], w_ref[0],
                       preferred_element_type=jnp.float32)


def _transform(emb_pad, ws):
  return pl.pallas_call(
      _transform_body,
      grid=(NPAIR, NP // 1024),
      in_specs=[
          pl.BlockSpec((1024, D), lambda s, j: (j, 0)),
          pl.BlockSpec((1, D, D), lambda s, j: (s, 0, 0)),
      ],
      out_specs=pl.BlockSpec((1, 1024, D), lambda s, j: (s, j, 0)),
      out_shape=jax.ShapeDtypeStruct((NPAIR, NP, D), jnp.float32),
  )(emb_pad, ws)


def _combine_body(g_ref, cnt_ref, b_ref, out_ref):
  c = cnt_ref[...]
  bmat = b_ref[...]
  nrows = c.shape[1]
  run = jnp.ones((nrows,), jnp.float32)
  acc = jnp.zeros((nrows, D), jnp.float32)
  for s in range(NPAIR - 1, -1, -1):
    cs = c[s]
    run = run * (2.0 / jnp.maximum(cs, 1.0))
    acc = acc + run[:, None] * g_ref[0, s] + (run * cs)[:, None] * bmat[s][None, :]
  out_ref[...] = acc


def _combine(g, cnt, bs):
  return pl.pallas_call(
      _combine_body,
      grid=(NP // 128,),
      in_specs=[
          pl.BlockSpec((1, NPAIR, 128, D),
                       lambda j: (0, 0, (j // 20) * 21 + j % 20, 0)),
          pl.BlockSpec((NPAIR, 128), lambda j: (0, (j // 20) * 21 + j % 20)),
          pl.BlockSpec((NPAIR, D), lambda j: (0, 0)),
      ],
      out_specs=pl.BlockSpec((128, D), lambda j: (j, 0)),
      out_shape=jax.ShapeDtypeStruct((NP, D), jnp.float32),
  )(g.reshape(1, NPAIR, 4 * QP, D), cnt, bs)


def kernel(edge_index, edge_type, embeddings, W0, b0):
  perm = jnp.array([0, 4, 1, 5, 2, 6, 3, 7], dtype=jnp.int32)
  ws = W0[perm]
  bs = b0[perm]

  emb_pad = jnp.zeros((NP, D), jnp.float32).at[:N_NODES].set(embeddings)
  t_tab = _transform(emb_pad, ws)
  t_rows = t_tab.reshape(NPAIR * NP, D)

  pad = EP - E
  a = jnp.concatenate([edge_index[0], jnp.zeros((pad,), jnp.int32)])
  b = jnp.concatenate([edge_index[1], jnp.zeros((pad,), jnp.int32)])
  t = jnp.concatenate([edge_type, jnp.full((pad,), -1, jnp.int32)])
  zrows = jnp.zeros((QPT, D), jnp.float32)

  g, cnt = _sc_aggregate(a, b, t, t_rows, zrows)
  out = _combine(g, cnt, bs)
  return out[:N_NODES]
